# baseline (speedup 1.0000x reference)
import jax
import jax.numpy as jnp
from jax import lax
from jax.experimental import pallas as pl
from jax.experimental.pallas import tpu as pltpu
from jax.experimental.pallas import tpu_sc as plsc

_NC = 2
_NS = 16
_NW = _NC * _NS
_L = 16
_ROWS = 4
_NSLOT = 4


def _sc_add_body(x_hbm, p_hbm, out_hbm, *scratch):
    B, S, D = x_hbm.shape
    p_bufs = scratch[0:_NSLOT]
    x_bufs = scratch[_NSLOT:2 * _NSLOT]
    sems = scratch[2 * _NSLOT:]
    sem_in = sems[0:_NSLOT]
    sem_out = sems[_NSLOT:]

    per_w = S // _NW
    n = per_w // _ROWS

    wid = lax.axis_index("s") * _NC + lax.axis_index("c")
    base = wid * per_w

    def mk_in(j, s):
        row0 = base + j * _ROWS
        return [
            pltpu.make_async_copy(
                p_hbm.at[pl.ds(row0, _ROWS), :], p_bufs[s], sem_in[s]),
            pltpu.make_async_copy(
                x_hbm.at[:, pl.ds(row0, _ROWS), :], x_bufs[s], sem_in[s]),
        ]

    def mk_out(j, s):
        row0 = base + j * _ROWS
        return [pltpu.make_async_copy(
            x_bufs[s], out_hbm.at[:, pl.ds(row0, _ROWS), :], sem_out[s])]

    def start_in(j, s):
        for cp in mk_in(j, s):
            cp.start()

    def wait_in(j, s):
        for cp in mk_in(j, s):
            cp.wait()

    def start_out(j, s):
        for cp in mk_out(j, s):
            cp.start()

    def wait_out(j, s):
        for cp in mk_out(j, s):
            cp.wait()

    def compute(s):
        pv_ref = p_bufs[s]
        xb = x_bufs[s]

        def add_body(r, carry):
            for c in range(D // _L):
                o = c * _L
                pv = pv_ref[r, pl.ds(o, _L)]
                for b in range(B):
                    xb[b, r, pl.ds(o, _L)] = xb[b, r, pl.ds(o, _L)] + pv
            return carry

        lax.fori_loop(0, _ROWS, add_body, 0)

    start_in(0, 0)
    start_in(1, 1)

    for j in range(_NSLOT):
        s = j % _NSLOT
        wait_in(j, s)
        compute(s)
        start_out(j, s)
        if j >= 2:
            wait_out(j - 2, (j - 2) % _NSLOT)
        start_in(j + 2, (j + 2) % _NSLOT)

    def trip(j0, carry):
        jb = j0 * _NSLOT
        for s in range(_NSLOT):
            j = jb + s
            wait_in(j, s)
            compute(s)
            start_out(j, s)
            wait_out(j - 2, (s + 2) % _NSLOT)
            start_in(j + 2, (s + 2) % _NSLOT)
        return carry

    lax.fori_loop(1, n // _NSLOT - 1, trip, 0)

    for j in range(n - _NSLOT, n):
        s = j % _NSLOT
        wait_in(j, s)
        compute(s)
        start_out(j, s)
        wait_out(j - 2, (j - 2) % _NSLOT)
        if j + 2 < n:
            start_in(j + 2, (j + 2) % _NSLOT)

    wait_out(n - 2, (n - 2) % _NSLOT)
    wait_out(n - 1, (n - 1) % _NSLOT)


def kernel(x, pos_table):
    B, S, D = x.shape
    scratch = []
    for _ in range(_NSLOT):
        scratch.append(pltpu.VMEM((_ROWS, D), jnp.float32))
    for _ in range(_NSLOT):
        scratch.append(pltpu.VMEM((B, _ROWS, D), jnp.float32))
    for _ in range(2 * _NSLOT):
        scratch.append(pltpu.SemaphoreType.DMA)

    sc_call = pl.kernel(
        _sc_add_body,
        out_type=jax.ShapeDtypeStruct((B, S, D), jnp.float32),
        mesh=plsc.VectorSubcoreMesh(core_axis_name="c", subcore_axis_name="s"),
        scratch_types=scratch,
    )
    return sc_call(x, pos_table)
